# baseline (speedup 1.0000x reference)
import jax
import jax.numpy as jnp
from jax.experimental import pallas as pl
from jax.experimental.pallas import tpu as pltpu

N = 10000
D = 128
ALPHA = 0.2
BM = 400


def _pass1_body(adj_ref, x_ref, w1_ref, w3_ref, b1_ref, s3_ref, s1_scr):
    @pl.when(pl.program_id(0) == 0)
    def _():
        s1_scr[...] = jnp.dot(
            x_ref[...], w1_ref[...], preferred_element_type=jnp.float32
        ).astype(jnp.bfloat16)

    a = adj_ref[...].astype(jnp.bfloat16)
    acc = jnp.dot(a, s1_scr[...], preferred_element_type=jnp.float32)
    acc = acc + b1_ref[...]
    u1 = jnp.where(acc > 0, acc, acc * ALPHA)
    s3_ref[...] = jnp.dot(
        u1.astype(jnp.bfloat16), w3_ref[...], preferred_element_type=jnp.float32
    ).astype(jnp.bfloat16)


def _pass2_body(adj_ref, x_ref, w2_ref, w4_ref, s3_ref, b2_ref, b3_ref,
                uo_ref, s4_ref, s2_scr):
    @pl.when(pl.program_id(0) == 0)
    def _():
        s2_scr[...] = jnp.dot(
            x_ref[...], w2_ref[...], preferred_element_type=jnp.float32
        ).astype(jnp.bfloat16)

    a = adj_ref[...].astype(jnp.bfloat16)
    acc2 = jnp.dot(a, s2_scr[...], preferred_element_type=jnp.float32)
    acc2 = acc2 + b2_ref[...]
    i1 = jnp.where(acc2 > 0, acc2, acc2 * ALPHA)
    s4_ref[...] = jnp.dot(
        i1.astype(jnp.bfloat16), w4_ref[...], preferred_element_type=jnp.float32
    ).astype(jnp.bfloat16)
    acc3 = jnp.dot(a, s3_ref[...], preferred_element_type=jnp.float32)
    uo_ref[...] = jnp.maximum(acc3 + b3_ref[...], 0.0)


def _pass3_body(adj_ref, s4_ref, b4_ref, io_ref):
    a = adj_ref[...].astype(jnp.bfloat16)
    acc = jnp.dot(a, s4_ref[...], preferred_element_type=jnp.float32)
    io_ref[...] = jnp.maximum(acc + b4_ref[...], 0.0)


_FULL = pl.BlockSpec((N, D), lambda i: (0, 0))
_ROW = pl.BlockSpec((1, D), lambda i: (0, 0))
_W = pl.BlockSpec((D, D), lambda i: (0, 0))
_ADJ = pl.BlockSpec((BM, N), lambda i: (i, 0))
_OUT = pl.BlockSpec((BM, D), lambda i: (i, 0))


def kernel(ufea, vfea, UV_adj, VU_adj, W1, W2, W3, W4, b1, b2, b3, b4):
    w3b = W3.astype(jnp.bfloat16)
    w4b = W4.astype(jnp.bfloat16)
    grid = (N // BM,)

    s3 = pl.pallas_call(
        _pass1_body,
        grid=grid,
        in_specs=[_ADJ, _FULL, _W, _W, _ROW],
        out_specs=_OUT,
        out_shape=jax.ShapeDtypeStruct((N, D), jnp.bfloat16),
        scratch_shapes=[pltpu.VMEM((N, D), jnp.bfloat16)],
    )(VU_adj, ufea, W1, w3b, b1.reshape(1, D))

    U_out, s4 = pl.pallas_call(
        _pass2_body,
        grid=grid,
        in_specs=[_ADJ, _FULL, _W, _W, _FULL, _ROW, _ROW],
        out_specs=[_OUT, _OUT],
        out_shape=[
            jax.ShapeDtypeStruct((N, D), jnp.float32),
            jax.ShapeDtypeStruct((N, D), jnp.bfloat16),
        ],
        scratch_shapes=[pltpu.VMEM((N, D), jnp.bfloat16)],
    )(UV_adj, vfea, W2, w4b, s3, b2.reshape(1, D), b3.reshape(1, D))

    I_out = pl.pallas_call(
        _pass3_body,
        grid=grid,
        in_specs=[_ADJ, _FULL, _ROW],
        out_specs=_OUT,
        out_shape=jax.ShapeDtypeStruct((N, D), jnp.float32),
    )(VU_adj, s4, b4.reshape(1, D))

    return U_out, I_out


# speedup vs baseline: 1.2662x; 1.2662x over previous
import jax
import jax.numpy as jnp
from jax.experimental import pallas as pl
from jax.experimental.pallas import tpu as pltpu

N = 10000
D = 128
ALPHA = 0.2
BM = 400


def _pass1_body(adj_ref, x_ref, w1_ref, w3_ref, b1_ref, s3_ref, s1_scr):
    @pl.when(pl.program_id(0) == 0)
    def _():
        s1_scr[...] = jnp.dot(
            x_ref[...], w1_ref[...], preferred_element_type=jnp.float32
        ).astype(jnp.bfloat16)

    a = adj_ref[...].astype(jnp.bfloat16)
    acc = jnp.dot(a, s1_scr[...], preferred_element_type=jnp.float32)
    acc = acc + b1_ref[...]
    u1 = jnp.where(acc > 0, acc, acc * ALPHA)
    s3_ref[...] = jnp.dot(
        u1.astype(jnp.bfloat16), w3_ref[...], preferred_element_type=jnp.float32
    ).astype(jnp.bfloat16)


def _pass2_body(adj_ref, x_ref, w2_ref, w4_ref, s3_ref, b2_ref, b3_ref,
                uo_ref, s4_ref, s23_scr):
    @pl.when(pl.program_id(0) == 0)
    def _():
        s23_scr[:, :D] = jnp.dot(
            x_ref[...], w2_ref[...], preferred_element_type=jnp.float32
        ).astype(jnp.bfloat16)
        s23_scr[:, D:] = s3_ref[...]

    a = adj_ref[...].astype(jnp.bfloat16)
    acc = jnp.dot(a, s23_scr[...], preferred_element_type=jnp.float32)
    acc2 = acc[:, :D] + b2_ref[...]
    i1 = jnp.where(acc2 > 0, acc2, acc2 * ALPHA)
    s4_ref[...] = jnp.dot(
        i1.astype(jnp.bfloat16), w4_ref[...], preferred_element_type=jnp.float32
    ).astype(jnp.bfloat16)
    uo_ref[...] = jnp.maximum(acc[:, D:] + b3_ref[...], 0.0)


def _pass3_body(adj_ref, s4_ref, b4_ref, io_ref):
    a = adj_ref[...].astype(jnp.bfloat16)
    acc = jnp.dot(a, s4_ref[...], preferred_element_type=jnp.float32)
    io_ref[...] = jnp.maximum(acc + b4_ref[...], 0.0)


_FULL = pl.BlockSpec((N, D), lambda i: (0, 0))
_ROW = pl.BlockSpec((1, D), lambda i: (0, 0))
_W = pl.BlockSpec((D, D), lambda i: (0, 0))
_ADJ = pl.BlockSpec((BM, N), lambda i: (i, 0))
_OUT = pl.BlockSpec((BM, D), lambda i: (i, 0))


def kernel(ufea, vfea, UV_adj, VU_adj, W1, W2, W3, W4, b1, b2, b3, b4):
    w3b = W3.astype(jnp.bfloat16)
    w4b = W4.astype(jnp.bfloat16)
    grid = (N // BM,)

    s3 = pl.pallas_call(
        _pass1_body,
        grid=grid,
        in_specs=[_ADJ, _FULL, _W, _W, _ROW],
        out_specs=_OUT,
        out_shape=jax.ShapeDtypeStruct((N, D), jnp.bfloat16),
        scratch_shapes=[pltpu.VMEM((N, D), jnp.bfloat16)],
    )(VU_adj, ufea, W1, w3b, b1.reshape(1, D))

    U_out, s4 = pl.pallas_call(
        _pass2_body,
        grid=grid,
        in_specs=[_ADJ, _FULL, _W, _W, _FULL, _ROW, _ROW],
        out_specs=[_OUT, _OUT],
        out_shape=[
            jax.ShapeDtypeStruct((N, D), jnp.float32),
            jax.ShapeDtypeStruct((N, D), jnp.bfloat16),
        ],
        scratch_shapes=[pltpu.VMEM((N, 2 * D), jnp.bfloat16)],
    )(UV_adj, vfea, W2, w4b, s3, b2.reshape(1, D), b3.reshape(1, D))

    I_out = pl.pallas_call(
        _pass3_body,
        grid=grid,
        in_specs=[_ADJ, _FULL, _ROW],
        out_specs=_OUT,
        out_shape=jax.ShapeDtypeStruct((N, D), jnp.float32),
    )(VU_adj, s4, b4.reshape(1, D))

    return U_out, I_out
